# baseline (speedup 1.0000x reference)
import functools

import jax
import jax.numpy as jnp
from jax import lax
from jax.experimental import pallas as pl
from jax.experimental.pallas import tpu as pltpu
from jax.experimental.pallas import tpu_sc as plsc

D_MODEL = 2048
N_CLUSTERS = 1024
CLUSTER_SIZE = 100
N_PROBES = 64
N_SEL = N_PROBES * CLUSTER_SIZE
NW = 32
ROWS_PER_W = N_SEL // NW
PAD = 8
BPW = ROWS_PER_W + PAD
N_PADDED = NW * BPW
CHUNK = 16
N_CHUNKS = BPW // CHUNK


def _select_body(h_ref, cent_ref, vmap_ref, idx_ref):
    h = h_ref[...]
    cent = cent_ref[...]
    norm = jnp.sqrt(jnp.sum(cent * cent, axis=0, keepdims=True))
    pre = cent / norm
    sims = jnp.dot(h, pre, preferred_element_type=jnp.float32)
    lane = lax.broadcasted_iota(jnp.int32, (1, N_CLUSTERS), 1)

    def body(i, scores):
        m = jnp.max(scores)
        c = jnp.min(jnp.where(scores == m, lane, N_CLUSTERS))
        idx_ref[pl.ds(i, 1)] = vmap_ref[pl.ds(c, 1)]
        return jnp.where(lane == c, -jnp.float32(1e30), scores)

    lax.fori_loop(0, N_PROBES, body, sims)


def _argmax_body(log_ref, idx_ref, out_ref):
    l = log_ref[...]
    ids = idx_ref[...]
    cols = l.shape[1]
    flat = (lax.broadcasted_iota(jnp.int32, l.shape, 0) * cols
            + lax.broadcasted_iota(jnp.int32, l.shape, 1))
    m = jnp.max(l)
    pos = jnp.min(jnp.where(l == m, flat, N_PADDED))
    val = jnp.sum(jnp.where(flat == pos, ids, 0))
    out_ref[...] = jnp.full((1, 1), val, jnp.int32)


def _logits_sc_body(w_hbm, idx_hbm, h_hbm, out_hbm,
                    idx_v, h_v, buf0, buf1, logits_v, sem0, sem1):
    wid = lax.axis_index("s") * 2 + lax.axis_index("c")
    base = wid * BPW
    pltpu.sync_copy(idx_hbm.at[pl.ds(base, BPW)], idx_v)
    pltpu.sync_copy(h_hbm, h_v)
    iota16 = lax.iota(jnp.int32, CHUNK)

    def start(c, buf, sem):
        idx_chunk = idx_v[pl.ds(c * CHUNK, CHUNK)]
        pltpu.make_async_copy(w_hbm.at[idx_chunk], buf, sem).start()

    def wait_for(c, buf, sem):
        idx_chunk = idx_v[pl.ds(c * CHUNK, CHUNK)]
        pltpu.make_async_copy(w_hbm.at[idx_chunk], buf, sem).wait()

    def compute(c, buf):
        def dbody(dc, acc):
            h16 = h_v[pl.ds(dc * 16, 16)]
            for j in range(16):
                d = dc * 16 + j
                col = plsc.load_gather(
                    buf, [iota16, jnp.full((CHUNK,), d, jnp.int32)])
                acc = acc + col * h16[j]
            return acc

        acc = lax.fori_loop(0, D_MODEL // 16, dbody,
                            jnp.zeros((CHUNK,), jnp.float32))
        logits_v[pl.ds(c * CHUNK, CHUNK)] = acc

    start(0, buf0, sem0)

    def pair(k, carry):
        c0 = 2 * k
        c1 = c0 + 1
        start(c1, buf1, sem1)
        wait_for(c0, buf0, sem0)
        compute(c0, buf0)
        start(c1 + 1, buf0, sem0)
        wait_for(c1, buf1, sem1)
        compute(c1, buf1)
        return carry

    lax.fori_loop(0, (N_CHUNKS - 1) // 2, pair, 0)
    wait_for(N_CHUNKS - 1, buf0, sem0)
    compute(N_CHUNKS - 1, buf0)
    pltpu.sync_copy(logits_v, out_hbm.at[pl.ds(base, BPW)])


@functools.lru_cache(maxsize=1)
def _build_logits_sc():
    mesh = plsc.VectorSubcoreMesh(core_axis_name="c", subcore_axis_name="s")
    return pl.kernel(
        _logits_sc_body,
        mesh=mesh,
        out_type=jax.ShapeDtypeStruct((N_PADDED,), jnp.float32),
        scratch_types=[
            pltpu.VMEM((BPW,), jnp.int32),
            pltpu.VMEM((D_MODEL,), jnp.float32),
            pltpu.VMEM((CHUNK, D_MODEL), jnp.float32),
            pltpu.VMEM((CHUNK, D_MODEL), jnp.float32),
            pltpu.VMEM((BPW,), jnp.float32),
            pltpu.SemaphoreType.DMA,
            pltpu.SemaphoreType.DMA,
        ],
        compiler_params=pltpu.CompilerParams(
            use_tc_tiling_on_sc=False, needs_layout_passes=False),
    )


def kernel(hidden_states, lm_head_weight, centroids, vocab_maps_tensor):
    h2d = hidden_states.reshape(1, D_MODEL)
    vmap3 = vocab_maps_tensor.reshape(N_CLUSTERS, 1, CLUSTER_SIZE)
    idx_sel = pl.pallas_call(
        _select_body,
        out_shape=jax.ShapeDtypeStruct((N_PROBES, 1, CLUSTER_SIZE), jnp.int32),
    )(h2d, centroids, vmap3)

    idx2 = idx_sel.reshape(NW, ROWS_PER_W)
    idx_pad = jnp.concatenate([idx2, idx2[:, :PAD]], axis=1).reshape(N_PADDED)

    logits = _build_logits_sc()(
        lm_head_weight, idx_pad, hidden_states.reshape(D_MODEL))

    out = pl.pallas_call(
        _argmax_body,
        out_shape=jax.ShapeDtypeStruct((1, 1), jnp.int32),
    )(logits.reshape(8, N_PADDED // 8), idx_pad.reshape(8, N_PADDED // 8))
    return out


# speedup vs baseline: 3.1803x; 3.1803x over previous
import functools

import jax
import jax.numpy as jnp
from jax import lax
from jax.experimental import pallas as pl
from jax.experimental.pallas import tpu as pltpu
from jax.experimental.pallas import tpu_sc as plsc

D_MODEL = 2048
N_CLUSTERS = 1024
CLUSTER_SIZE = 100
N_PROBES = 64
N_SEL = N_PROBES * CLUSTER_SIZE
NW = 32
ROWS_PER_W = N_SEL // NW
PAD = 8
BPW = ROWS_PER_W + PAD
N_PADDED = NW * BPW
CHUNK = 16
N_CHUNKS = BPW // CHUNK


def _select_body(h_ref, cent_ref, vmap_ref, idx_ref):
    h = h_ref[...]
    cent = cent_ref[...]
    norm = jnp.sqrt(jnp.sum(cent * cent, axis=0, keepdims=True))
    pre = cent / norm
    sims = jnp.dot(h, pre, preferred_element_type=jnp.float32)
    lane = lax.broadcasted_iota(jnp.int32, (1, N_CLUSTERS), 1)

    def body(i, scores):
        m = jnp.max(scores)
        c = jnp.min(jnp.where(scores == m, lane, N_CLUSTERS))
        idx_ref[pl.ds(i, 1)] = vmap_ref[pl.ds(c, 1)]
        return jnp.where(lane == c, -jnp.float32(1e30), scores)

    lax.fori_loop(0, N_PROBES, body, sims)


def _argmax_body(log_ref, idx_ref, out_ref):
    l = log_ref[...]
    ids = idx_ref[...]
    cols = l.shape[1]
    flat = (lax.broadcasted_iota(jnp.int32, l.shape, 0) * cols
            + lax.broadcasted_iota(jnp.int32, l.shape, 1))
    m = jnp.max(l)
    pos = jnp.min(jnp.where(l == m, flat, N_PADDED))
    val = jnp.sum(jnp.where(flat == pos, ids, 0))
    out_ref[...] = jnp.full((1, 1), val, jnp.int32)


def _logits_sc_body(w_hbm, idx_hbm, h_hbm, out_hbm,
                    idx_v, h_v, buf0, buf1, logits_v, sem0, sem1):
    wid = lax.axis_index("s") * 2 + lax.axis_index("c")
    base = wid * BPW
    pltpu.sync_copy(idx_hbm.at[pl.ds(base, BPW)], idx_v)
    pltpu.sync_copy(h_hbm, h_v)
    iota16 = lax.iota(jnp.int32, CHUNK)

    def start(c, buf, sem):
        idx_chunk = idx_v[pl.ds(c * CHUNK, CHUNK)]
        pltpu.make_async_copy(w_hbm.at[idx_chunk], buf, sem).start()

    def wait_for(c, buf, sem):
        idx_chunk = idx_v[pl.ds(c * CHUNK, CHUNK)]
        pltpu.make_async_copy(w_hbm.at[idx_chunk], buf, sem).wait()

    def compute(c, buf):
        def dbody(dc, acc):
            h16 = h_v[pl.ds(dc * 16, 16)]
            for j in range(16):
                d = dc * 16 + j
                col = plsc.load_gather(
                    buf, [iota16, jnp.full((CHUNK,), d, jnp.int32)])
                acc = acc + col * h16[j]
            return acc

        acc = lax.fori_loop(0, D_MODEL // 16, dbody,
                            jnp.zeros((CHUNK,), jnp.float32))
        logits_v[pl.ds(c * CHUNK, CHUNK)] = acc

    start(0, buf0, sem0)

    def pair(k, carry):
        c0 = 2 * k
        c1 = c0 + 1
        start(c1, buf1, sem1)
        wait_for(c0, buf0, sem0)
        compute(c0, buf0)
        start(c1 + 1, buf0, sem0)
        wait_for(c1, buf1, sem1)
        compute(c1, buf1)
        return carry

    lax.fori_loop(0, (N_CHUNKS - 1) // 2, pair, 0)
    wait_for(N_CHUNKS - 1, buf0, sem0)
    compute(N_CHUNKS - 1, buf0)
    pltpu.sync_copy(logits_v, out_hbm.at[pl.ds(base, BPW)])


@functools.lru_cache(maxsize=1)
def _build_logits_sc():
    mesh = plsc.VectorSubcoreMesh(core_axis_name="c", subcore_axis_name="s")
    return pl.kernel(
        _logits_sc_body,
        mesh=mesh,
        out_type=jax.ShapeDtypeStruct((N_PADDED,), jnp.float32),
        scratch_types=[
            pltpu.VMEM((BPW,), jnp.int32),
            pltpu.VMEM((D_MODEL,), jnp.float32),
            pltpu.VMEM((CHUNK, D_MODEL), jnp.float32),
            pltpu.VMEM((CHUNK, D_MODEL), jnp.float32),
            pltpu.VMEM((BPW,), jnp.float32),
            pltpu.SemaphoreType.DMA,
            pltpu.SemaphoreType.DMA,
        ],
        compiler_params=pltpu.CompilerParams(needs_layout_passes=False),
    )


def kernel(hidden_states, lm_head_weight, centroids, vocab_maps_tensor):
    h2d = hidden_states.reshape(1, D_MODEL)
    vmap3 = vocab_maps_tensor.reshape(N_CLUSTERS, 1, CLUSTER_SIZE)
    idx_sel = pl.pallas_call(
        _select_body,
        out_shape=jax.ShapeDtypeStruct((N_PROBES, 1, CLUSTER_SIZE), jnp.int32),
    )(h2d, centroids, vmap3)

    idx2 = idx_sel.reshape(NW, ROWS_PER_W)
    idx_pad = jnp.concatenate([idx2, idx2[:, :PAD]], axis=1).reshape(N_PADDED)

    logits = _build_logits_sc()(
        lm_head_weight, idx_pad, hidden_states.reshape(D_MODEL))

    out = pl.pallas_call(
        _argmax_body,
        out_shape=jax.ShapeDtypeStruct((1, 1), jnp.int32),
    )(logits.reshape(8, N_PADDED // 8), idx_pad.reshape(8, N_PADDED // 8))
    return out


# speedup vs baseline: 9.8518x; 3.0978x over previous
import functools

import jax
import jax.numpy as jnp
from jax import lax
from jax.experimental import pallas as pl
from jax.experimental.pallas import tpu as pltpu
from jax.experimental.pallas import tpu_sc as plsc

D_MODEL = 2048
N_CLUSTERS = 1024
CLUSTER_SIZE = 100
N_PROBES = 64
N_SEL = N_PROBES * CLUSTER_SIZE
NW = 32
ROWS_PER_W = N_SEL // NW
PAD = 8
BPW = ROWS_PER_W + PAD
N_PADDED = NW * BPW
CHUNK = 16
N_CHUNKS = BPW // CHUNK


def _select_body(h_ref, cent_ref, vmap_ref, idx_ref):
    h = h_ref[...]
    cent = cent_ref[...]
    norm = jnp.sqrt(jnp.sum(cent * cent, axis=0, keepdims=True))
    pre = cent / norm
    sims = jnp.dot(h, pre, preferred_element_type=jnp.float32)
    lane = lax.broadcasted_iota(jnp.int32, (1, N_CLUSTERS), 1)

    def body(i, scores):
        m = jnp.max(scores)
        c = jnp.min(jnp.where(scores == m, lane, N_CLUSTERS))
        idx_ref[pl.ds(i, 1)] = vmap_ref[pl.ds(c, 1)]
        return jnp.where(lane == c, -jnp.float32(1e30), scores)

    lax.fori_loop(0, N_PROBES, body, sims)


def _argmax_body(log_ref, idx_ref, out_ref):
    l = log_ref[...]
    ids = idx_ref[...]
    cols = l.shape[1]
    flat = (lax.broadcasted_iota(jnp.int32, l.shape, 0) * cols
            + lax.broadcasted_iota(jnp.int32, l.shape, 1))
    m = jnp.max(l)
    pos = jnp.min(jnp.where(l == m, flat, N_PADDED))
    val = jnp.sum(jnp.where(flat == pos, ids, 0))
    out_ref[...] = jnp.full((1, 1), val, jnp.int32)


def _logits_sc_body(w_hbm, idx_hbm, h_hbm, out_hbm,
                    idx_v, h_v, buf0, buf1, logits_v, sem0, sem1):
    wid = lax.axis_index("s") * 2 + lax.axis_index("c")
    base = wid * BPW
    pltpu.sync_copy(idx_hbm.at[pl.ds(base, BPW)], idx_v)
    pltpu.sync_copy(h_hbm, h_v)
    iota16 = lax.iota(jnp.int32, CHUNK)

    def start(c, buf, sem):
        idx_chunk = idx_v[pl.ds(c * CHUNK, CHUNK)]
        pltpu.make_async_copy(w_hbm.at[idx_chunk], buf, sem).start()

    def wait_for(c, buf, sem):
        idx_chunk = idx_v[pl.ds(c * CHUNK, CHUNK)]
        pltpu.make_async_copy(w_hbm.at[idx_chunk], buf, sem).wait()

    def compute(c, buf):
        def dbody(dc, accs):
            h16 = h_v[pl.ds(dc * 16, 16)]
            return tuple(
                accs[r] + buf[r, pl.ds(dc * 16, 16)] * h16
                for r in range(CHUNK))

        accs = lax.fori_loop(
            0, D_MODEL // 16, dbody,
            tuple(jnp.zeros((16,), jnp.float32) for _ in range(CHUNK)))
        res = jnp.zeros((16,), jnp.float32)
        for r in range(CHUNK):
            res = jnp.where(iota16 == r, jnp.sum(accs[r]), res)
        logits_v[pl.ds(c * CHUNK, CHUNK)] = res

    start(0, buf0, sem0)

    def pair(k, carry):
        c0 = 2 * k
        c1 = c0 + 1
        start(c1, buf1, sem1)
        wait_for(c0, buf0, sem0)
        compute(c0, buf0)
        start(c1 + 1, buf0, sem0)
        wait_for(c1, buf1, sem1)
        compute(c1, buf1)
        return carry

    lax.fori_loop(0, (N_CHUNKS - 1) // 2, pair, 0)
    wait_for(N_CHUNKS - 1, buf0, sem0)
    compute(N_CHUNKS - 1, buf0)
    pltpu.sync_copy(logits_v, out_hbm.at[pl.ds(base, BPW)])


@functools.lru_cache(maxsize=1)
def _build_logits_sc():
    mesh = plsc.VectorSubcoreMesh(core_axis_name="c", subcore_axis_name="s")
    return pl.kernel(
        _logits_sc_body,
        mesh=mesh,
        out_type=jax.ShapeDtypeStruct((N_PADDED,), jnp.float32),
        scratch_types=[
            pltpu.VMEM((BPW,), jnp.int32),
            pltpu.VMEM((D_MODEL,), jnp.float32),
            pltpu.VMEM((CHUNK, D_MODEL), jnp.float32),
            pltpu.VMEM((CHUNK, D_MODEL), jnp.float32),
            pltpu.VMEM((BPW,), jnp.float32),
            pltpu.SemaphoreType.DMA,
            pltpu.SemaphoreType.DMA,
        ],
        compiler_params=pltpu.CompilerParams(needs_layout_passes=False),
    )


def kernel(hidden_states, lm_head_weight, centroids, vocab_maps_tensor):
    h2d = hidden_states.reshape(1, D_MODEL)
    vmap3 = vocab_maps_tensor.reshape(N_CLUSTERS, 1, CLUSTER_SIZE)
    idx_sel = pl.pallas_call(
        _select_body,
        out_shape=jax.ShapeDtypeStruct((N_PROBES, 1, CLUSTER_SIZE), jnp.int32),
    )(h2d, centroids, vmap3)

    idx2 = idx_sel.reshape(NW, ROWS_PER_W)
    idx_pad = jnp.concatenate([idx2, idx2[:, :PAD]], axis=1).reshape(N_PADDED)

    logits = _build_logits_sc()(
        lm_head_weight, idx_pad, hidden_states.reshape(D_MODEL))

    out = pl.pallas_call(
        _argmax_body,
        out_shape=jax.ShapeDtypeStruct((1, 1), jnp.int32),
    )(logits.reshape(8, N_PADDED // 8), idx_pad.reshape(8, N_PADDED // 8))
    return out
